# baseline (speedup 1.0000x reference)
import functools
import math

import jax
import jax.numpy as jnp
from jax.experimental import pallas as pl
from jax.experimental.pallas import tpu as pltpu

NUM_GROUPS = 32
EPS = 1e-6


def _one_batch(x, w_ref, vecs, mg, attn_scale, inv_count, out_dtype):
    C = x.shape[0]
    gamma = vecs[:, 0:1]
    beta = vecs[:, 1:2]

    s1 = jnp.sum(x, axis=1, keepdims=True)
    s2 = jnp.sum(x * x, axis=1, keepdims=True)
    g1 = jax.lax.dot_general(mg, s1, (((0,), (0,)), ((), ())),
                             preferred_element_type=jnp.float32)
    g2 = jax.lax.dot_general(mg, s2, (((0,), (0,)), ((), ())),
                             preferred_element_type=jnp.float32)
    mean_g = g1 * inv_count
    var_g = g2 * inv_count - mean_g * mean_g
    mean_c = jnp.dot(mg, mean_g, preferred_element_type=jnp.float32)
    var_c = jnp.dot(mg, var_g, preferred_element_type=jnp.float32)
    inv_std = jax.lax.rsqrt(var_c + EPS)
    scale = inv_std * gamma
    shift = beta - mean_c * scale

    xb = x.astype(jnp.bfloat16)

    def proj(w_bf, sc, sh, b):
        ws = w_bf * sc.astype(jnp.bfloat16)
        bs = jax.lax.dot_general(w_bf, sh.astype(jnp.bfloat16),
                                 (((0,), (0,)), ((), ())),
                                 preferred_element_type=jnp.float32) + b
        return jax.lax.dot_general(
            ws, xb, (((0,), (0,)), ((), ())),
            preferred_element_type=jnp.float32) + bs

    scale_q = scale * attn_scale
    shift_q = shift * attn_scale
    bq = vecs[:, 2:3] * attn_scale
    q = proj(w_ref[0], scale_q, shift_q, bq).astype(jnp.bfloat16)
    k = proj(w_ref[1], scale, shift, vecs[:, 3:4]).astype(jnp.bfloat16)
    v = proj(w_ref[2], scale, shift, vecs[:, 4:5]).astype(jnp.bfloat16)

    s = jax.lax.dot_general(q, k, (((0,), (0,)), ((), ())),
                            preferred_element_type=jnp.float32)
    pf = jnp.exp(s)
    l = jnp.sum(pf, axis=1, keepdims=True)
    p = pf.astype(jnp.bfloat16)
    lt = pl.reciprocal(l, approx=True).T

    o = jax.lax.dot_general(v, p, (((1,), (1,)), ((), ())),
                            preferred_element_type=jnp.float32) * lt
    pr = jax.lax.dot_general(
        w_ref[3], o.astype(jnp.bfloat16), (((0,), (0,)), ((), ())),
        preferred_element_type=jnp.float32) + vecs[:, 5:6]

    return (x + pr).astype(out_dtype)


def _fused_attn_kernel(x_ref, w_ref, vecs_ref, mg_ref, o_ref,
                       *, attn_scale, inv_count):
    o_ref[0] = _one_batch(x_ref[0], w_ref, vecs_ref[...], mg_ref[...],
                          attn_scale, inv_count, o_ref.dtype)


def kernel(x, gamma, beta, wq_t, bq, wk_t, bk, wv_t, bv, wp_t, bp):
    B, C, H, W = x.shape
    N = H * W
    G = NUM_GROUPS

    x3 = x.reshape(B, C, N)
    w = jnp.stack([wq_t, wk_t, wv_t, wp_t]).astype(jnp.bfloat16)
    vecs = jnp.concatenate(
        [gamma, beta, bq, bk, bv, bp,
         jnp.zeros((2, C), jnp.float32)], axis=0).T
    mg = (jnp.arange(C)[:, None] // (C // G)
          == jnp.arange(G)[None, :]).astype(jnp.float32)

    out = pl.pallas_call(
        functools.partial(_fused_attn_kernel,
                          attn_scale=1.0 / math.sqrt(C),
                          inv_count=1.0 / float(N * (C // G))),
        out_shape=jax.ShapeDtypeStruct((B, C, N), x.dtype),
        grid=(B,),
        in_specs=[
            pl.BlockSpec((1, C, N), lambda b: (b, 0, 0)),
            pl.BlockSpec((4, C, C), lambda b: (0, 0, 0)),
            pl.BlockSpec((C, 8), lambda b: (0, 0)),
            pl.BlockSpec((C, G), lambda b: (0, 0)),
        ],
        out_specs=pl.BlockSpec((1, C, N), lambda b: (b, 0, 0)),
        compiler_params=pltpu.CompilerParams(
            dimension_semantics=("parallel",),
            allow_input_fusion=(True, True, True, True),
            vmem_limit_bytes=60 * 1024 * 1024),
    )(x3, w, vecs, mg)

    return out.reshape(B, C, H, W)


# speedup vs baseline: 1.0071x; 1.0071x over previous
import functools
import math

import jax
import jax.numpy as jnp
from jax.experimental import pallas as pl
from jax.experimental.pallas import tpu as pltpu

NUM_GROUPS = 32
EPS = 1e-6


def _one_batch(x, w_ref, vecs, mg, attn_scale, inv_count, out_dtype):
    C = x.shape[0]
    gamma = vecs[:, 0:1]
    beta = vecs[:, 1:2]

    s1 = jnp.sum(x, axis=1, keepdims=True)
    s2 = jnp.sum(x * x, axis=1, keepdims=True)
    g1 = jax.lax.dot_general(mg, s1, (((0,), (0,)), ((), ())),
                             preferred_element_type=jnp.float32)
    g2 = jax.lax.dot_general(mg, s2, (((0,), (0,)), ((), ())),
                             preferred_element_type=jnp.float32)
    mean_g = g1 * inv_count
    var_g = g2 * inv_count - mean_g * mean_g
    mean_c = jnp.dot(mg, mean_g, preferred_element_type=jnp.float32)
    var_c = jnp.dot(mg, var_g, preferred_element_type=jnp.float32)
    inv_std = jax.lax.rsqrt(var_c + EPS)
    scale = inv_std * gamma
    shift = beta - mean_c * scale

    xb = x.astype(jnp.bfloat16)

    def proj(w_bf, sc, sh, b):
        ws = w_bf * sc.astype(jnp.bfloat16)
        bs = jax.lax.dot_general(w_bf, sh.astype(jnp.bfloat16),
                                 (((0,), (0,)), ((), ())),
                                 preferred_element_type=jnp.float32) + b
        return jax.lax.dot_general(
            ws, xb, (((0,), (0,)), ((), ())),
            preferred_element_type=jnp.float32) + bs

    scale_q = scale * attn_scale
    shift_q = shift * attn_scale
    bq = vecs[:, 2:3] * attn_scale
    q = proj(w_ref[0], scale_q, shift_q, bq).astype(jnp.bfloat16)
    k = proj(w_ref[1], scale, shift, vecs[:, 3:4]).astype(jnp.bfloat16)
    v = proj(w_ref[2], scale, shift, vecs[:, 4:5]).astype(jnp.bfloat16)

    s = jax.lax.dot_general(q, k, (((0,), (0,)), ((), ())),
                            preferred_element_type=jnp.float32)
    pf = jnp.exp(s)
    l = jnp.sum(pf, axis=1, keepdims=True)
    p = pf.astype(jnp.bfloat16)
    lt = pl.reciprocal(l, approx=True).T

    o = jax.lax.dot_general(v, p, (((1,), (1,)), ((), ())),
                            preferred_element_type=jnp.float32) * lt
    pr = jax.lax.dot_general(
        w_ref[3], o.astype(jnp.bfloat16), (((0,), (0,)), ((), ())),
        preferred_element_type=jnp.float32) + vecs[:, 5:6]

    return (x + pr).astype(out_dtype)


def _fused_attn_kernel(x_ref, w_ref, vecs_ref, mg_ref, o_ref,
                       *, attn_scale, inv_count):
    o_ref[0] = _one_batch(x_ref[0], w_ref, vecs_ref[...], mg_ref[...],
                          attn_scale, inv_count, o_ref.dtype)


def kernel(x, gamma, beta, wq_t, bq, wk_t, bk, wv_t, bv, wp_t, bp):
    B, C, H, W = x.shape
    N = H * W
    G = NUM_GROUPS

    x3 = x.reshape(B, C, N)
    w = jnp.stack([wq_t, wk_t, wv_t, wp_t]).astype(jnp.bfloat16)
    vecs = jnp.concatenate(
        [gamma, beta, bq, bk, bv, bp,
         jnp.zeros((2, C), jnp.float32)], axis=0).T
    mg = (jnp.arange(C)[:, None] // (C // G)
          == jnp.arange(G)[None, :]).astype(jnp.float32)

    out = pl.pallas_call(
        functools.partial(_fused_attn_kernel,
                          attn_scale=1.0 / math.sqrt(C),
                          inv_count=1.0 / float(N * (C // G))),
        out_shape=jax.ShapeDtypeStruct((B, C, N), x.dtype),
        grid=(B,),
        in_specs=[
            pl.BlockSpec((1, C, N), lambda b: (b, 0, 0)),
            pl.BlockSpec((4, C, C), lambda b: (0, 0, 0)),
            pl.BlockSpec((C, 8), lambda b: (0, 0)),
            pl.BlockSpec((C, G), lambda b: (0, 0)),
        ],
        out_specs=pl.BlockSpec((1, C, N), lambda b: (b, 0, 0)),
        compiler_params=pltpu.CompilerParams(
            dimension_semantics=("parallel",),
            vmem_limit_bytes=60 * 1024 * 1024),
    )(x3, w, vecs, mg)

    return out.reshape(B, C, H, W)


# speedup vs baseline: 1.0157x; 1.0085x over previous
import functools
import math

import jax
import jax.numpy as jnp
from jax.experimental import pallas as pl
from jax.experimental.pallas import tpu as pltpu

NUM_GROUPS = 32
EPS = 1e-6


def _one_batch(x, w_ref, vecs, mg, attn_scale, inv_count, out_dtype):
    C = x.shape[0]
    gamma = vecs[:, 0:1]
    beta = vecs[:, 1:2]

    s1 = jnp.sum(x, axis=1, keepdims=True)
    s2 = jnp.sum(x * x, axis=1, keepdims=True)
    g1 = jax.lax.dot_general(mg, s1, (((0,), (0,)), ((), ())),
                             preferred_element_type=jnp.float32)
    g2 = jax.lax.dot_general(mg, s2, (((0,), (0,)), ((), ())),
                             preferred_element_type=jnp.float32)
    mean_g = g1 * inv_count
    var_g = g2 * inv_count - mean_g * mean_g
    mean_c = jnp.dot(mg, mean_g, preferred_element_type=jnp.float32)
    var_c = jnp.dot(mg, var_g, preferred_element_type=jnp.float32)
    inv_std = jax.lax.rsqrt(var_c + EPS)
    scale = inv_std * gamma
    shift = beta - mean_c * scale

    xb = x.astype(jnp.bfloat16)

    def proj(w_bf, sc, sh, b):
        ws = w_bf * sc.astype(jnp.bfloat16)
        bs = jax.lax.dot_general(w_bf, sh.astype(jnp.bfloat16),
                                 (((0,), (0,)), ((), ())),
                                 preferred_element_type=jnp.float32) + b
        return jax.lax.dot_general(
            ws, xb, (((0,), (0,)), ((), ())),
            preferred_element_type=jnp.float32) + bs

    scale_q = scale * attn_scale
    shift_q = shift * attn_scale
    bq = vecs[:, 2:3] * attn_scale
    q = proj(w_ref[0], scale_q, shift_q, bq).astype(jnp.bfloat16)
    k = proj(w_ref[1], scale, shift, vecs[:, 3:4]).astype(jnp.bfloat16)
    v = proj(w_ref[2], scale, shift, vecs[:, 4:5]).astype(jnp.bfloat16)

    s = jax.lax.dot_general(q, k, (((0,), (0,)), ((), ())),
                            preferred_element_type=jnp.float32)
    pf = jnp.exp(s)
    l = jnp.sum(pf, axis=1, keepdims=True)
    p = pf.astype(jnp.bfloat16)
    lt = pl.reciprocal(l, approx=True).T

    o = jax.lax.dot_general(v, p, (((1,), (1,)), ((), ())),
                            preferred_element_type=jnp.float32) * lt
    pr = jax.lax.dot_general(
        w_ref[3], o.astype(jnp.bfloat16), (((0,), (0,)), ((), ())),
        preferred_element_type=jnp.float32) + vecs[:, 5:6]

    return (x + pr).astype(out_dtype)


def _fused_attn_kernel(x_ref, w_ref, vecs_ref, mg_ref, o_ref,
                       *, attn_scale, inv_count):
    o_ref[0] = _one_batch(x_ref[0], w_ref, vecs_ref[...], mg_ref[...],
                          attn_scale, inv_count, o_ref.dtype)


def kernel(x, gamma, beta, wq_t, bq, wk_t, bk, wv_t, bv, wp_t, bp):
    B, C, H, W = x.shape
    N = H * W
    G = NUM_GROUPS

    x3 = x.reshape(B, C, N)
    w = jnp.stack([wq_t, wk_t, wv_t, wp_t]).astype(jnp.bfloat16)
    vecs = jnp.stack([gamma[0], beta[0], bq[0], bk[0], bv[0], bp[0]],
                     axis=1)
    mg = (jnp.arange(C)[:, None] // (C // G)
          == jnp.arange(G)[None, :]).astype(jnp.float32)

    out = pl.pallas_call(
        functools.partial(_fused_attn_kernel,
                          attn_scale=1.0 / math.sqrt(C),
                          inv_count=1.0 / float(N * (C // G))),
        out_shape=jax.ShapeDtypeStruct((B, C, N), x.dtype),
        grid=(B,),
        in_specs=[
            pl.BlockSpec((1, C, N), lambda b: (b, 0, 0)),
            pl.BlockSpec((4, C, C), lambda b: (0, 0, 0)),
            pl.BlockSpec((C, 6), lambda b: (0, 0)),
            pl.BlockSpec((C, G), lambda b: (0, 0)),
        ],
        out_specs=pl.BlockSpec((1, C, N), lambda b: (b, 0, 0)),
        compiler_params=pltpu.CompilerParams(
            dimension_semantics=("parallel",),
            vmem_limit_bytes=60 * 1024 * 1024),
    )(x3, w, vecs, mg)

    return out.reshape(B, C, H, W)
